# baseline (speedup 1.0000x reference)
import functools
import jax, jax.numpy as jnp
from jax import lax
from jax.experimental import pallas as pl
from jax.experimental.pallas import tpu as pltpu
from jax.experimental.pallas import tpu_sc as plsc

V = 1_000_000
D = 64
S = 200
B = 4096
NTILE = 16
NPLANE_PER_SC = D // 2
UNITS = S * 2
UPT = UNITS // NTILE
UTOK = B // 2
VSLICE = 62496
VTAIL = V - NTILE * VSLICE


def _body(ids_hbm, table_hbm, out_hbm, idx_v, gbuf, plane_sh, gsem, lsem):
    c = lax.axis_index("c")
    t = lax.axis_index("s")

    pltpu.sync_copy(ids_hbm.at[t], idx_v)

    def do_plane(jj, carry):
        j = c * NPLANE_PER_SC + jj
        pltpu.async_copy(
            table_hbm.at[j, pl.ds(t * VSLICE, VSLICE)],
            plane_sh.at[pl.ds(t * VSLICE, VSLICE)],
            lsem,
        ).wait()
        @pl.when(t == NTILE - 1)
        def _():
            pltpu.async_copy(
                table_hbm.at[j, pl.ds(NTILE * VSLICE, VTAIL)],
                plane_sh.at[pl.ds(NTILE * VSLICE, VTAIL)],
                lsem,
            ).wait()
        plsc.subcore_barrier()
        def do_unit(k, carry2):
            u = t * UPT + k
            s = u // 2
            half = u % 2
            pltpu.async_copy(plane_sh.at[idx_v.at[k]], gbuf, gsem).wait()
            pltpu.sync_copy(gbuf, out_hbm.at[s, j, pl.ds(half * UTOK, UTOK)])
            return carry2
        lax.fori_loop(0, UPT, do_unit, 0)
        plsc.subcore_barrier()
        return carry

    lax.fori_loop(0, NPLANE_PER_SC, do_plane, 0)


def kernel(token_ids, weight):
    ids_op = token_ids.T.astype(jnp.int32).reshape(NTILE, UPT, UTOK)
    table_t = weight.T
    mesh = plsc.VectorSubcoreMesh(core_axis_name="c", subcore_axis_name="s")
    run = pl.kernel(
        _body,
        out_type=jax.ShapeDtypeStruct((S, D, B), jnp.float32),
        mesh=mesh,
        scratch_types=[
            pltpu.VMEM((UPT, UTOK), jnp.int32),
            pltpu.VMEM((UTOK,), jnp.float32),
            pltpu.VMEM_SHARED((V,), jnp.float32),
            pltpu.SemaphoreType.DMA,
            pltpu.SemaphoreType.DMA,
        ],
        compiler_params=pltpu.CompilerParams(use_tc_tiling_on_sc=False),
    )
    out_t = run(ids_op, table_t)
    return out_t.transpose(2, 0, 1)


# speedup vs baseline: 3.0288x; 3.0288x over previous
import jax, jax.numpy as jnp
from jax import lax
from jax.experimental import pallas as pl
from jax.experimental.pallas import tpu as pltpu
from jax.experimental.pallas import tpu_sc as plsc

NW = 32
SEQ = 200
BBLK = 128
D = 64


def _body(ids_hbm, table_hbm, out_hbm, idx_v, g0, g1, t0, t1,
          gsem0, gsem1, ssem0, ssem1):
    c = lax.axis_index("c")
    sub = lax.axis_index("s")
    w = sub * 2 + c
    pltpu.sync_copy(ids_hbm.at[w], idx_v)

    iota16 = lax.iota(jnp.int32, 16)

    def gather(s, gbuf, gsem):
        return pltpu.async_copy(table_hbm.at[idx_v.at[s]], gbuf, gsem)

    def wait_gather(gbuf, gsem):
        pltpu.make_async_copy(table_hbm.at[idx_v.at[0]], gbuf, gsem).wait()

    def store(s, tbuf, ssem):
        return pltpu.async_copy(
            tbuf, out_hbm.at[s, :, w, :, :], ssem)

    def wait_store(tbuf, ssem):
        pltpu.make_async_copy(tbuf, out_hbm.at[0, :, w, :, :], ssem).wait()

    def transpose(gbuf, tbuf):
        for g in range(8):
            tau = iota16 + (g * 16)
            for f in range(D):
                fv = jnp.full((16,), f, jnp.int32)
                vals = plsc.load_gather(gbuf, [tau, fv])
                tbuf[f // 8, f % 8, pl.ds(g * 16, 16)] = vals

    gather(0, g0, gsem0)

    def pair(i, carry):
        s0 = 2 * i
        s1 = s0 + 1
        @pl.when(s1 < SEQ)
        def _():
            gather(s1, g1, gsem1)
        wait_gather(g0, gsem0)
        @pl.when(i > 0)
        def _():
            wait_store(t0, ssem0)
        transpose(g0, t0)
        store(s0, t0, ssem0)
        @pl.when(s0 + 2 < SEQ)
        def _():
            gather(s0 + 2, g0, gsem0)
        wait_gather(g1, gsem1)
        @pl.when(i > 0)
        def _():
            wait_store(t1, ssem1)
        transpose(g1, t1)
        store(s1, t1, ssem1)
        return carry

    lax.fori_loop(0, SEQ // 2, pair, 0)
    wait_store(t0, ssem0)
    wait_store(t1, ssem1)


def kernel(token_ids, weight):
    ids_op = (token_ids.T.astype(jnp.int32)
              .reshape(SEQ, NW, BBLK).transpose(1, 0, 2))
    mesh = plsc.VectorSubcoreMesh(core_axis_name="c", subcore_axis_name="s")
    run = pl.kernel(
        _body,
        out_type=jax.ShapeDtypeStruct((SEQ, 8, NW, 8, BBLK), jnp.float32),
        mesh=mesh,
        scratch_types=[
            pltpu.VMEM((SEQ, BBLK), jnp.int32),
            pltpu.VMEM((BBLK, D), jnp.float32),
            pltpu.VMEM((BBLK, D), jnp.float32),
            pltpu.VMEM((8, 8, BBLK), jnp.float32),
            pltpu.VMEM((8, 8, BBLK), jnp.float32),
            pltpu.SemaphoreType.DMA,
            pltpu.SemaphoreType.DMA,
            pltpu.SemaphoreType.DMA,
            pltpu.SemaphoreType.DMA,
        ],
        compiler_params=pltpu.CompilerParams(use_tc_tiling_on_sc=False, needs_layout_passes=False),
    )
    out5 = run(ids_op, weight)
    return (out5.transpose(2, 4, 0, 1, 3)
            .reshape(4096, SEQ, D))


# speedup vs baseline: 4.5353x; 1.4974x over previous
import jax, jax.numpy as jnp
from jax import lax
from jax.experimental import pallas as pl
from jax.experimental.pallas import tpu as pltpu
from jax.experimental.pallas import tpu_sc as plsc

NW = 32
SEQ = 200
BBLK = 128
D = 64


def _body(ids_hbm, table_hbm, out_hbm, idx_v, g0, g1, t0, t1,
          gsem0, gsem1, ssem0, ssem1):
    c = lax.axis_index("c")
    sub = lax.axis_index("s")
    w = sub * 2 + c
    pltpu.sync_copy(ids_hbm.at[w], idx_v)

    iota16 = lax.iota(jnp.int32, 16)

    def gather(s, gbuf, gsem):
        return pltpu.async_copy(table_hbm.at[idx_v.at[s]], gbuf, gsem)

    def wait_gather(gbuf, gsem):
        pltpu.make_async_copy(table_hbm.at[idx_v.at[0]], gbuf, gsem).wait()

    def store(s, tbuf, ssem):
        return pltpu.async_copy(
            tbuf, out_hbm.at[s, :, w, :, :], ssem)

    def wait_store(tbuf, ssem):
        pltpu.make_async_copy(tbuf, out_hbm.at[0, :, w, :, :], ssem).wait()

    def transpose(gbuf, tbuf):
        K = 16
        for g in range(8):
            tau = iota16 + (g * 16)
            for f0 in range(0, D, K):
                vals = [
                    plsc.load_gather(gbuf, [tau, jnp.full((16,), f0 + k, jnp.int32)])
                    for k in range(K)
                ]
                for k in range(K):
                    f = f0 + k
                    tbuf[f // 8, f % 8, pl.ds(g * 16, 16)] = vals[k]

    gather(0, g0, gsem0)

    def pair(i, carry):
        s0 = 2 * i
        s1 = s0 + 1
        @pl.when(s1 < SEQ)
        def _():
            gather(s1, g1, gsem1)
        wait_gather(g0, gsem0)
        @pl.when(i > 0)
        def _():
            wait_store(t0, ssem0)
        transpose(g0, t0)
        store(s0, t0, ssem0)
        @pl.when(s0 + 2 < SEQ)
        def _():
            gather(s0 + 2, g0, gsem0)
        wait_gather(g1, gsem1)
        @pl.when(i > 0)
        def _():
            wait_store(t1, ssem1)
        transpose(g1, t1)
        store(s1, t1, ssem1)
        return carry

    lax.fori_loop(0, SEQ // 2, pair, 0)
    wait_store(t0, ssem0)
    wait_store(t1, ssem1)


def kernel(token_ids, weight):
    ids_op = (token_ids.T.astype(jnp.int32)
              .reshape(SEQ, NW, BBLK).transpose(1, 0, 2))
    mesh = plsc.VectorSubcoreMesh(core_axis_name="c", subcore_axis_name="s")
    run = pl.kernel(
        _body,
        out_type=jax.ShapeDtypeStruct((SEQ, 8, NW, 8, BBLK), jnp.float32),
        mesh=mesh,
        scratch_types=[
            pltpu.VMEM((SEQ, BBLK), jnp.int32),
            pltpu.VMEM((BBLK, D), jnp.float32),
            pltpu.VMEM((BBLK, D), jnp.float32),
            pltpu.VMEM((8, 8, BBLK), jnp.float32),
            pltpu.VMEM((8, 8, BBLK), jnp.float32),
            pltpu.SemaphoreType.DMA,
            pltpu.SemaphoreType.DMA,
            pltpu.SemaphoreType.DMA,
            pltpu.SemaphoreType.DMA,
        ],
        compiler_params=pltpu.CompilerParams(use_tc_tiling_on_sc=False, needs_layout_passes=False),
    )
    out5 = run(ids_op, weight)
    return (out5.transpose(2, 4, 0, 1, 3)
            .reshape(4096, SEQ, D))


# speedup vs baseline: 4.5370x; 1.0004x over previous
import jax, jax.numpy as jnp
from jax import lax
from jax.experimental import pallas as pl
from jax.experimental.pallas import tpu as pltpu
from jax.experimental.pallas import tpu_sc as plsc

NW = 32
SEQ = 200
BBLK = 128
D = 64


def _body(ids_hbm, table_hbm, out_hbm, idx_v, g0, g1, t0, t1,
          gsem0, gsem1, ssem0, ssem1):
    c = lax.axis_index("c")
    sub = lax.axis_index("s")
    w = sub * 2 + c
    pltpu.sync_copy(ids_hbm.at[w], idx_v)

    iota16 = lax.iota(jnp.int32, 16)

    def gather(s, gbuf, gsem):
        return pltpu.async_copy(table_hbm.at[idx_v.at[s]], gbuf, gsem)

    def wait_gather(gbuf, gsem):
        pltpu.make_async_copy(table_hbm.at[idx_v.at[0]], gbuf, gsem).wait()

    def store(s, tbuf, ssem):
        return pltpu.async_copy(
            tbuf, out_hbm.at[s, :, w, :, :], ssem)

    def wait_store(tbuf, ssem):
        pltpu.make_async_copy(tbuf, out_hbm.at[0, :, w, :, :], ssem).wait()

    def transpose(gbuf, tbuf):
        for g in range(8):
            tau = iota16 + (g * 16)
            for d0 in range(0, D, 8):
                batch = []
                for k in range(8):
                    fv = (tau + (d0 + k)) & 63
                    vals = plsc.load_gather(gbuf, [tau, fv])
                    batch.append((fv, vals))
                for fv, vals in batch:
                    plsc.store_scatter(
                        tbuf,
                        [lax.shift_right_logical(fv, 3), fv & 7, tau],
                        vals)

    gather(0, g0, gsem0)

    def pair(i, carry):
        s0 = 2 * i
        s1 = s0 + 1
        @pl.when(s1 < SEQ)
        def _():
            gather(s1, g1, gsem1)
        wait_gather(g0, gsem0)
        @pl.when(i > 0)
        def _():
            wait_store(t0, ssem0)
        transpose(g0, t0)
        store(s0, t0, ssem0)
        @pl.when(s0 + 2 < SEQ)
        def _():
            gather(s0 + 2, g0, gsem0)
        wait_gather(g1, gsem1)
        @pl.when(i > 0)
        def _():
            wait_store(t1, ssem1)
        transpose(g1, t1)
        store(s1, t1, ssem1)
        return carry

    lax.fori_loop(0, SEQ // 2, pair, 0)
    wait_store(t0, ssem0)
    wait_store(t1, ssem1)


def kernel(token_ids, weight):
    ids_op = (token_ids.T.astype(jnp.int32)
              .reshape(SEQ, NW, BBLK).transpose(1, 0, 2))
    mesh = plsc.VectorSubcoreMesh(core_axis_name="c", subcore_axis_name="s")
    run = pl.kernel(
        _body,
        out_type=jax.ShapeDtypeStruct((SEQ, 8, NW, 8, BBLK), jnp.float32),
        mesh=mesh,
        scratch_types=[
            pltpu.VMEM((SEQ, BBLK), jnp.int32),
            pltpu.VMEM((BBLK, D), jnp.float32),
            pltpu.VMEM((BBLK, D), jnp.float32),
            pltpu.VMEM((8, 8, BBLK), jnp.float32),
            pltpu.VMEM((8, 8, BBLK), jnp.float32),
            pltpu.SemaphoreType.DMA,
            pltpu.SemaphoreType.DMA,
            pltpu.SemaphoreType.DMA,
            pltpu.SemaphoreType.DMA,
        ],
        compiler_params=pltpu.CompilerParams(use_tc_tiling_on_sc=False, needs_layout_passes=False),
    )
    out5 = run(ids_op, weight)
    return (out5.transpose(2, 4, 0, 1, 3)
            .reshape(4096, SEQ, D))
